# baseline (speedup 1.0000x reference)
import functools

import jax
import jax.numpy as jnp
from jax import lax
from jax.experimental import pallas as pl
from jax.experimental.pallas import tpu as pltpu
from jax.experimental.pallas import tpu_sc as plsc

N = 10000
D = 128
K = 16
E = N * K

CB = 200
NCB = N // CB

RB = 400
NRB = N // RB

NB = 200
EB = NB * K
NNB = N // NB

_NC, _NS = 2, 16
_NW = _NC * _NS
CHUNK = 128
PER_W = -(-E // (_NW * CHUNK)) * CHUNK
E_PAD = PER_W * _NW
NCHUNK = PER_W // CHUNK


def _bc_body(x_ref, wcat_ref, b1_ref, b_ref, c_ref):
    bc = jnp.dot(x_ref[...], wcat_ref[...], preferred_element_type=jnp.float32)
    b_ref[...] = bc[:, :D]
    c_ref[...] = bc[:, D:] + b1_ref[...]


def _knn_body(cpos_ref, posT_ref, out_ref):
    i = pl.program_id(0)
    cpos = cpos_ref[...]
    pT = posT_ref[...]
    q = jnp.dot(cpos.astype(jnp.bfloat16), pT.astype(jnp.bfloat16),
                preferred_element_type=jnp.float32)
    csq = jnp.sum(cpos * cpos, axis=1, keepdims=True)
    asq = jnp.sum(pT * pT, axis=0, keepdims=True)
    d2 = (csq - 2.0 * q) + asq
    cols = lax.broadcasted_iota(jnp.int32, (CB, N), 1)
    selfidx = i * CB + lax.broadcasted_iota(jnp.int32, (CB, 1), 0)
    inf = jnp.float32(jnp.inf)
    big = jnp.int32(0x7FFFFFFF)
    d2 = jnp.where(cols == selfidx, inf, d2)
    picks = []
    pending = []
    for r in range(K):
        cond = None
        for t in pending:
            c = cols == t
            cond = c if cond is None else (cond | c)
        if cond is None:
            m = jnp.min(d2, axis=1, keepdims=True)
            sel = d2 == m
        else:
            m = jnp.min(jnp.where(cond, inf, d2), axis=1, keepdims=True)
            sel = (d2 == m) & (~cond)
        idx = jnp.min(jnp.where(sel, cols, big), axis=1, keepdims=True)
        picks.append(idx)
        pending.append(idx)
        if len(pending) == 4 and r < K - 1:
            cond = ((cols == pending[0]) | (cols == pending[1])
                    | (cols == pending[2]) | (cols == pending[3]))
            d2 = jnp.where(cond, inf, d2)
            pending = []
    out_ref[...] = jnp.concatenate(picks, axis=1)


def _mlp_body(bg_ref, c_ref, w2_ref, b2_ref, out_ref):
    bg = bg_ref[...].reshape(NB, K, D)
    c = c_ref[...]
    h = jnp.maximum(bg + c[:, None, :], 0.0).reshape(EB, D)
    p = jnp.dot(h, w2_ref[...], preferred_element_type=jnp.float32)
    out_ref[...] = jnp.max(p.reshape(NB, K, D), axis=1) + b2_ref[...]


def _gather_fn():
    mesh = plsc.VectorSubcoreMesh(core_axis_name="c", subcore_axis_name="s")

    @functools.partial(
        pl.kernel,
        mesh=mesh,
        out_type=jax.ShapeDtypeStruct((E_PAD, D), jnp.float32),
        scratch_types=[
            pltpu.VMEM((CHUNK,), jnp.int32),
            pltpu.VMEM((CHUNK,), jnp.int32),
            pltpu.VMEM((CHUNK, D), jnp.float32),
            pltpu.VMEM((CHUNK, D), jnp.float32),
            pltpu.SemaphoreType.DMA,
            pltpu.SemaphoreType.DMA,
            pltpu.SemaphoreType.DMA,
            pltpu.SemaphoreType.DMA,
            pltpu.SemaphoreType.DMA,
            pltpu.SemaphoreType.DMA,
        ],
    )
    def gather_k(table_hbm, idx_hbm, out_hbm,
                 idx_v0, idx_v1, rows_v0, rows_v1,
                 sem_i0, sem_i1, sem_g0, sem_g1, sem_s0, sem_s1):
        wid = lax.axis_index("s") * _NC + lax.axis_index("c")
        base = wid * PER_W

        def idx_start(ci, buf, sem):
            off = base + ci * CHUNK
            return pltpu.make_async_copy(
                idx_hbm.at[pl.ds(off, CHUNK)], buf, sem)

        idx_start(0, idx_v0, sem_i0).start()
        idx_start(1, idx_v1, sem_i1).start()

        def body(j, carry):
            ci0 = 2 * j
            ci1 = 2 * j + 1
            idx_start(ci0, idx_v0, sem_i0).wait()
            g0 = pltpu.make_async_copy(table_hbm.at[idx_v0], rows_v0, sem_g0)
            g0.start()
            idx_start(ci1, idx_v1, sem_i1).wait()
            g1 = pltpu.make_async_copy(table_hbm.at[idx_v1], rows_v1, sem_g1)
            g1.start()
            g0.wait()
            s0 = pltpu.make_async_copy(
                rows_v0, out_hbm.at[pl.ds(base + ci0 * CHUNK, CHUNK)], sem_s0)
            s0.start()
            idx_start(ci0 + 2, idx_v0, sem_i0).start()
            g1.wait()
            s1 = pltpu.make_async_copy(
                rows_v1, out_hbm.at[pl.ds(base + ci1 * CHUNK, CHUNK)], sem_s1)
            s1.start()
            idx_start(ci1 + 2, idx_v1, sem_i1).start()
            s0.wait()
            s1.wait()
            return carry

        lax.fori_loop(0, NCHUNK // 2, body, 0)
        idx_start(NCHUNK, idx_v0, sem_i0).wait()
        idx_start(NCHUNK + 1, idx_v1, sem_i1).wait()

    return gather_k


def kernel(x, pos, W1, b1, W2, b2):
    wcat = jnp.concatenate([W1[D:], W1[:D] - W1[D:]], axis=1)
    b1r = b1.reshape(1, D)
    b2r = b2.reshape(1, D)

    bmat, cmat = pl.pallas_call(
        _bc_body,
        grid=(NRB,),
        in_specs=[
            pl.BlockSpec((RB, D), lambda i: (i, 0)),
            pl.BlockSpec((D, 2 * D), lambda i: (0, 0)),
            pl.BlockSpec((1, D), lambda i: (0, 0)),
        ],
        out_specs=[
            pl.BlockSpec((RB, D), lambda i: (i, 0)),
            pl.BlockSpec((RB, D), lambda i: (i, 0)),
        ],
        out_shape=[
            jax.ShapeDtypeStruct((N, D), jnp.float32),
            jax.ShapeDtypeStruct((N, D), jnp.float32),
        ],
    )(x, wcat, b1r)

    nbr = pl.pallas_call(
        _knn_body,
        grid=(NCB,),
        in_specs=[
            pl.BlockSpec((CB, 3), lambda i: (i, 0)),
            pl.BlockSpec((3, N), lambda i: (0, 0)),
        ],
        out_specs=pl.BlockSpec((CB, K), lambda i: (i, 0)),
        out_shape=jax.ShapeDtypeStruct((N, K), jnp.int32),
    )(pos, pos.T)

    col = nbr.reshape(-1)
    idx_pad = jnp.concatenate(
        [col, jnp.zeros((E_PAD + 2 * CHUNK - E,), jnp.int32)])

    bg = _gather_fn()(bmat, idx_pad)

    out = pl.pallas_call(
        _mlp_body,
        grid=(NNB,),
        in_specs=[
            pl.BlockSpec((EB, D), lambda i: (i, 0)),
            pl.BlockSpec((NB, D), lambda i: (i, 0)),
            pl.BlockSpec((D, D), lambda i: (0, 0)),
            pl.BlockSpec((1, D), lambda i: (0, 0)),
        ],
        out_specs=pl.BlockSpec((NB, D), lambda i: (i, 0)),
        out_shape=jax.ShapeDtypeStruct((N, D), jnp.float32),
    )(bg, cmat, W2, b2r)
    return out


# speedup vs baseline: 1.7249x; 1.7249x over previous
import functools

import jax
import jax.numpy as jnp
from jax import lax
from jax.experimental import pallas as pl
from jax.experimental.pallas import tpu as pltpu
from jax.experimental.pallas import tpu_sc as plsc

N = 10000
D = 128
K = 16
E = N * K

CB = 200
NCB = N // CB

RB = 400
NRB = N // RB

NB = 200
EB = NB * K
NNB = N // NB

_NC, _NS = 2, 16
_NW = _NC * _NS
CHUNK = 128
PER_W = -(-E // (_NW * CHUNK)) * CHUNK
E_PAD = PER_W * _NW
NCHUNK = PER_W // CHUNK


def _bc_body(x_ref, wcat_ref, b1_ref, b_ref, c_ref):
    bc = jnp.dot(x_ref[...], wcat_ref[...], preferred_element_type=jnp.float32)
    b_ref[...] = bc[:, :D]
    c_ref[...] = bc[:, D:] + b1_ref[...]


def _knn_body(cpos_ref, posT_ref, out_ref):
    i = pl.program_id(0)
    cpos = cpos_ref[...]
    pT = posT_ref[...]
    q = jnp.dot(cpos.astype(jnp.bfloat16), pT.astype(jnp.bfloat16),
                preferred_element_type=jnp.float32)
    csq = jnp.sum(cpos * cpos, axis=1, keepdims=True)
    asq = jnp.sum(pT * pT, axis=0, keepdims=True)
    d2 = (csq - 2.0 * q) + asq
    cols = lax.broadcasted_iota(jnp.int32, (CB, N), 1)
    selfidx = i * CB + lax.broadcasted_iota(jnp.int32, (CB, 1), 0)
    inf = jnp.float32(jnp.inf)
    big = jnp.int32(0x7FFFFFFF)
    d2 = jnp.where(cols == selfidx, inf, d2)
    picks = []
    for r in range(K):
        m = jnp.min(d2, axis=1, keepdims=True)
        idx = jnp.min(jnp.where(d2 == m, cols, big), axis=1, keepdims=True)
        picks.append(idx)
        if r < K - 1:
            d2 = jnp.where(cols == idx, inf, d2)
    out_ref[...] = jnp.concatenate(picks, axis=1)


def _mlp_body(bg_ref, c_ref, w2_ref, b2_ref, out_ref):
    bg = bg_ref[...].reshape(NB, K, D)
    c = c_ref[...]
    h = jnp.maximum(bg + c[:, None, :], 0.0).reshape(EB, D)
    p = jnp.dot(h, w2_ref[...], preferred_element_type=jnp.float32)
    out_ref[...] = jnp.max(p.reshape(NB, K, D), axis=1) + b2_ref[...]


def _gather_fn():
    mesh = plsc.VectorSubcoreMesh(core_axis_name="c", subcore_axis_name="s")

    @functools.partial(
        pl.kernel,
        mesh=mesh,
        out_type=jax.ShapeDtypeStruct((E_PAD, D), jnp.float32),
        scratch_types=[
            pltpu.VMEM((CHUNK,), jnp.int32),
            pltpu.VMEM((CHUNK,), jnp.int32),
            pltpu.VMEM((CHUNK, D), jnp.float32),
            pltpu.VMEM((CHUNK, D), jnp.float32),
            pltpu.SemaphoreType.DMA,
            pltpu.SemaphoreType.DMA,
            pltpu.SemaphoreType.DMA,
            pltpu.SemaphoreType.DMA,
            pltpu.SemaphoreType.DMA,
            pltpu.SemaphoreType.DMA,
        ],
    )
    def gather_k(table_hbm, idx_hbm, out_hbm,
                 idx_v0, idx_v1, rows_v0, rows_v1,
                 sem_i0, sem_i1, sem_g0, sem_g1, sem_s0, sem_s1):
        wid = lax.axis_index("s") * _NC + lax.axis_index("c")
        base = wid * PER_W

        def idx_start(ci, buf, sem):
            off = base + ci * CHUNK
            return pltpu.make_async_copy(
                idx_hbm.at[pl.ds(off, CHUNK)], buf, sem)

        idx_start(0, idx_v0, sem_i0).start()
        idx_start(1, idx_v1, sem_i1).start()

        def body(j, carry):
            ci0 = 2 * j
            ci1 = 2 * j + 1
            idx_start(ci0, idx_v0, sem_i0).wait()
            g0 = pltpu.make_async_copy(table_hbm.at[idx_v0], rows_v0, sem_g0)
            g0.start()
            idx_start(ci1, idx_v1, sem_i1).wait()
            g1 = pltpu.make_async_copy(table_hbm.at[idx_v1], rows_v1, sem_g1)
            g1.start()
            g0.wait()
            s0 = pltpu.make_async_copy(
                rows_v0, out_hbm.at[pl.ds(base + ci0 * CHUNK, CHUNK)], sem_s0)
            s0.start()
            idx_start(ci0 + 2, idx_v0, sem_i0).start()
            g1.wait()
            s1 = pltpu.make_async_copy(
                rows_v1, out_hbm.at[pl.ds(base + ci1 * CHUNK, CHUNK)], sem_s1)
            s1.start()
            idx_start(ci1 + 2, idx_v1, sem_i1).start()
            s0.wait()
            s1.wait()
            return carry

        lax.fori_loop(0, NCHUNK // 2, body, 0)
        idx_start(NCHUNK, idx_v0, sem_i0).wait()
        idx_start(NCHUNK + 1, idx_v1, sem_i1).wait()

    return gather_k


def kernel(x, pos, W1, b1, W2, b2):
    wcat = jnp.concatenate([W1[D:], W1[:D] - W1[D:]], axis=1)
    b1r = b1.reshape(1, D)
    b2r = b2.reshape(1, D)

    bmat, cmat = pl.pallas_call(
        _bc_body,
        grid=(NRB,),
        in_specs=[
            pl.BlockSpec((RB, D), lambda i: (i, 0)),
            pl.BlockSpec((D, 2 * D), lambda i: (0, 0)),
            pl.BlockSpec((1, D), lambda i: (0, 0)),
        ],
        out_specs=[
            pl.BlockSpec((RB, D), lambda i: (i, 0)),
            pl.BlockSpec((RB, D), lambda i: (i, 0)),
        ],
        out_shape=[
            jax.ShapeDtypeStruct((N, D), jnp.float32),
            jax.ShapeDtypeStruct((N, D), jnp.float32),
        ],
    )(x, wcat, b1r)

    nbr = pl.pallas_call(
        _knn_body,
        grid=(NCB,),
        in_specs=[
            pl.BlockSpec((CB, 3), lambda i: (i, 0)),
            pl.BlockSpec((3, N), lambda i: (0, 0)),
        ],
        out_specs=pl.BlockSpec((CB, K), lambda i: (i, 0)),
        out_shape=jax.ShapeDtypeStruct((N, K), jnp.int32),
    )(pos, pos.T)

    col = nbr.reshape(-1)
    idx_pad = jnp.concatenate(
        [col, jnp.zeros((E_PAD + 2 * CHUNK - E,), jnp.int32)])

    bg = _gather_fn()(bmat, idx_pad)

    out = pl.pallas_call(
        _mlp_body,
        grid=(NNB,),
        in_specs=[
            pl.BlockSpec((EB, D), lambda i: (i, 0)),
            pl.BlockSpec((NB, D), lambda i: (i, 0)),
            pl.BlockSpec((D, D), lambda i: (0, 0)),
            pl.BlockSpec((1, D), lambda i: (0, 0)),
        ],
        out_specs=pl.BlockSpec((NB, D), lambda i: (i, 0)),
        out_shape=jax.ShapeDtypeStruct((N, D), jnp.float32),
    )(bg, cmat, W2, b2r)
    return out
